# baseline (speedup 1.0000x reference)
import functools

import jax
import jax.numpy as jnp
from jax import lax
from jax.experimental import pallas as pl
from jax.experimental.pallas import tpu as pltpu
from jax.experimental.pallas import tpu_sc as plsc

N = 10000
E = 320000
D = 128
G = 100
NW = 32
NSUB = 16
CHUNK = 128
NSPLIT = 4
HALF = CHUNK // NSPLIT
CH = 80
EPW = CH * CHUNK
EPAD = NW * EPW
RPT = (-(-N // NSUB) + 127) // 128 * 128
NPAD = RPT * NSUB


def _sc_aggregate(table, srcs, dsts, zeros2d, zeros1d, ones1d):
    mesh = plsc.VectorSubcoreMesh(core_axis_name="c", subcore_axis_name="s")

    @functools.partial(
        pl.kernel,
        out_type=(jax.ShapeDtypeStruct((2, NPAD, D), jnp.float32),
                  jax.ShapeDtypeStruct((2 * NPAD,), jnp.float32)),
        mesh=mesh,
        scratch_types=[
            pltpu.VMEM((CH // 2, CHUNK), jnp.int32),
            pltpu.VMEM((CH // 2, CHUNK), jnp.int32),
            [pltpu.VMEM((CHUNK, D), jnp.float32) for _ in range(2)],
            pltpu.VMEM((CHUNK,), jnp.float32),
            [[pltpu.SemaphoreType.DMA for _ in range(NSPLIT)]
             for _ in range(2)],
            pltpu.VMEM_SHARED((NPAD, D), jnp.float32),
            pltpu.VMEM_SHARED((NPAD,), jnp.float32),
        ],
    )
    def agg_kernel(table_hbm, srcs_hbm, dsts_hbm, zeros2d_hbm, zeros1d_hbm,
                   ones1d_hbm, out_hbm, deg_hbm,
                   src_idx, dst_idx, rows, ones_v, sems, acc, dacc):

        def start_gather(c, b):
            for p in range(NSPLIT):
                pltpu.async_copy(
                    table_hbm.at[src_idx.at[c, pl.ds(p * HALF, HALF)]],
                    rows[b].at[pl.ds(p * HALF, HALF)], sems[b][p])

        def wait_gather(c, b):
            for p in range(NSPLIT):
                pltpu.make_async_copy(
                    table_hbm.at[src_idx.at[c, pl.ds(p * HALF, HALF)]],
                    rows[b].at[pl.ds(p * HALF, HALF)], sems[b][p]).wait()

        cid = lax.axis_index("c")
        sid = lax.axis_index("s")
        wid = cid * NSUB + sid

        CHH = CH // 2
        for half in range(2):
            base = wid * CH + half * CHH
            pltpu.sync_copy(srcs_hbm.at[pl.ds(base, CHH)], src_idx)
            pltpu.sync_copy(dsts_hbm.at[pl.ds(base, CHH)], dst_idx)
            for b in range(2):
                start_gather(b, b)
            if half == 0:
                pltpu.sync_copy(zeros2d_hbm, acc.at[pl.ds(sid * RPT, RPT)])
                pltpu.sync_copy(zeros1d_hbm, dacc.at[pl.ds(sid * RPT, RPT)])
                pltpu.sync_copy(ones1d_hbm, ones_v)
                plsc.subcore_barrier()

            def body(g, carry):
                for b in range(2):
                    c = 2 * g + b
                    wait_gather(c, b)
                    pltpu.sync_copy(rows[b], acc.at[dst_idx.at[c]], add=True)
                    pltpu.sync_copy(ones_v, dacc.at[dst_idx.at[c]], add=True)
                    start_gather(c + 2, b)
                return carry

            lax.fori_loop(0, CHH // 2 - 1, body, 0)
            for b in range(2):
                c = CHH - 2 + b
                wait_gather(c, b)
                pltpu.sync_copy(rows[b], acc.at[dst_idx.at[c]], add=True)
                pltpu.sync_copy(ones_v, dacc.at[dst_idx.at[c]], add=True)
        plsc.subcore_barrier()
        pltpu.sync_copy(acc.at[pl.ds(sid * RPT, RPT)],
                        out_hbm.at[cid, pl.ds(sid * RPT, RPT)])
        pltpu.sync_copy(dacc.at[pl.ds(sid * RPT, RPT)],
                        deg_hbm.at[pl.ds(cid * NPAD + sid * RPT, RPT)])

    return agg_kernel(table, srcs, dsts, zeros2d, zeros1d, ones1d)


def _tc_stage(feats, p0, p1, d0, d1, w_self, w_neigh, b_sage,
              w1, b1, g1, be1, w2, b2, g2, be2, w3, b3):
    blk = 800
    nblk = -(-N // blk)
    npg = N // G
    gpb = blk // npg

    def body(f_ref, p0_ref, p1_ref, d0_ref, d1_ref, ws_ref, wn_ref, b_ref,
             w1_ref, b1_ref, g1_ref, be1_ref, w2_ref, b2_ref, g2_ref,
             be2_ref, w3_ref, b3_ref, o_ref, hg_s):
        i = pl.program_id(0)

        @pl.when(i < nblk)
        def _node():
            agg = p0_ref[...] + p1_ref[...]
            deg = jnp.maximum(d0_ref[...] + d1_ref[...], 1.0)
            h_neigh = agg / deg
            t = jnp.dot(h_neigh, wn_ref[...],
                        preferred_element_type=jnp.float32)
            h = (jnp.dot(f_ref[...], ws_ref[...],
                         preferred_element_type=jnp.float32)
                 + t + b_ref[...])
            h = jnp.maximum(h, 0.0)
            hg_s[pl.ds(i * gpb, gpb), :] = jnp.max(
                h.reshape(gpb, npg, D), axis=1)

        @pl.when(i == nblk)
        def _head():
            hg = hg_s[...][:G, :]
            z = jnp.maximum(
                jnp.dot(hg, w1_ref[...], preferred_element_type=jnp.float32)
                + b1_ref[...], 0.0)
            m = jnp.mean(z, axis=0, keepdims=True)
            v = jnp.mean((z - m) ** 2, axis=0, keepdims=True)
            z = g1_ref[...] * (z - m) * lax.rsqrt(v + 1e-5) + be1_ref[...]
            z = jnp.maximum(
                jnp.dot(z, w2_ref[...], preferred_element_type=jnp.float32)
                + b2_ref[...], 0.0)
            m = jnp.mean(z, axis=0, keepdims=True)
            v = jnp.mean((z - m) ** 2, axis=0, keepdims=True)
            z = g2_ref[...] * (z - m) * lax.rsqrt(v + 1e-5) + be2_ref[...]
            o_ref[...] = (jnp.dot(z, w3_ref[...],
                                  preferred_element_type=jnp.float32)
                          + b3_ref[...])

    clamp = lambda i: (jnp.minimum(i, nblk - 1), 0)
    const = lambda i: (0, 0)
    return pl.pallas_call(
        body,
        grid=(nblk + 1,),
        in_specs=[
            pl.BlockSpec((blk, D), clamp),
            pl.BlockSpec((blk, D), clamp),
            pl.BlockSpec((blk, D), clamp),
            pl.BlockSpec((blk, 1), clamp),
            pl.BlockSpec((blk, 1), clamp),
            pl.BlockSpec((D, D), const),
            pl.BlockSpec((D, D), const),
            pl.BlockSpec((1, D), const),
            pl.BlockSpec((D, D), const),
            pl.BlockSpec((1, D), const),
            pl.BlockSpec((1, D), const),
            pl.BlockSpec((1, D), const),
            pl.BlockSpec((D, 64), const),
            pl.BlockSpec((1, 64), const),
            pl.BlockSpec((1, 64), const),
            pl.BlockSpec((1, 64), const),
            pl.BlockSpec((64, 1), const),
            pl.BlockSpec((1, 1), const),
        ],
        out_specs=pl.BlockSpec((G, 1), const),
        out_shape=jax.ShapeDtypeStruct((G, 1), jnp.float32),
        scratch_shapes=[pltpu.VMEM((nblk * gpb, D), jnp.float32)],
    )(feats, p0, p1, d0, d1, w_self, w_neigh, b_sage,
      w1, b1, g1, be1, w2, b2, g2, be2, w3, b3)


def kernel(feats, edge_index, node_graph_ids, W_self, W_neigh, b_sage,
           W1, b1, g1, be1, W2, b2, g2, be2, W3, b3):
    src = edge_index[0]
    dst = edge_index[1]
    pad_i = jnp.arange(EPAD - E, dtype=jnp.int32)
    srcs = jnp.concatenate(
        [src, pad_i % N]).reshape(NW * CH, CHUNK)
    dsts = jnp.concatenate(
        [dst, N + pad_i % (NPAD - N)]).reshape(NW * CH, CHUNK)
    zeros2d = jnp.zeros((RPT, D), jnp.float32)
    zeros1d = jnp.zeros((RPT,), jnp.float32)
    ones1d = jnp.ones((CHUNK,), jnp.float32)

    parts, degs = _sc_aggregate(feats, srcs, dsts, zeros2d, zeros1d, ones1d)

    d2 = degs.reshape(2, NPAD, 1)
    return _tc_stage(feats, parts[0], parts[1], d2[0], d2[1], W_self, W_neigh,
                     b_sage.reshape(1, D), W1, b1.reshape(1, -1),
                     g1.reshape(1, -1), be1.reshape(1, -1), W2,
                     b2.reshape(1, -1), g2.reshape(1, -1), be2.reshape(1, -1),
                     W3, b3.reshape(1, -1))


# speedup vs baseline: 1.0086x; 1.0086x over previous
import functools

import jax
import jax.numpy as jnp
from jax import lax
from jax.experimental import pallas as pl
from jax.experimental.pallas import tpu as pltpu
from jax.experimental.pallas import tpu_sc as plsc

N = 10000
E = 320000
D = 128
G = 100
NW = 32
NSUB = 16
CHUNK = 128
NSPLIT = 2
HALF = CHUNK // NSPLIT
CH = 80
EPW = CH * CHUNK
EPAD = NW * EPW
RPT = (-(-N // NSUB) + 127) // 128 * 128
NPAD = RPT * NSUB


def _sc_aggregate(table, srcs, dsts, zeros2d, zeros1d, ones1d):
    mesh = plsc.VectorSubcoreMesh(core_axis_name="c", subcore_axis_name="s")

    @functools.partial(
        pl.kernel,
        out_type=(jax.ShapeDtypeStruct((2, NPAD, D), jnp.float32),
                  jax.ShapeDtypeStruct((2 * NPAD,), jnp.float32)),
        mesh=mesh,
        scratch_types=[
            pltpu.VMEM((CH // 2, CHUNK), jnp.int32),
            pltpu.VMEM((CH // 2, CHUNK), jnp.int32),
            [pltpu.VMEM((CHUNK, D), jnp.float32) for _ in range(2)],
            pltpu.VMEM((CHUNK,), jnp.float32),
            [[pltpu.SemaphoreType.DMA for _ in range(NSPLIT)]
             for _ in range(2)],
            pltpu.VMEM_SHARED((NPAD, D), jnp.float32),
            pltpu.VMEM_SHARED((NPAD,), jnp.float32),
        ],
    )
    def agg_kernel(table_hbm, srcs_hbm, dsts_hbm, zeros2d_hbm, zeros1d_hbm,
                   ones1d_hbm, out_hbm, deg_hbm,
                   src_idx, dst_idx, rows, ones_v, sems, acc, dacc):

        def start_gather(c, b):
            for p in range(NSPLIT):
                pltpu.async_copy(
                    table_hbm.at[src_idx.at[c, pl.ds(p * HALF, HALF)]],
                    rows[b].at[pl.ds(p * HALF, HALF)], sems[b][p])

        def wait_gather(c, b):
            for p in range(NSPLIT):
                pltpu.make_async_copy(
                    table_hbm.at[src_idx.at[c, pl.ds(p * HALF, HALF)]],
                    rows[b].at[pl.ds(p * HALF, HALF)], sems[b][p]).wait()

        cid = lax.axis_index("c")
        sid = lax.axis_index("s")
        wid = cid * NSUB + sid

        CHH = CH // 2
        for half in range(2):
            base = wid * CH + half * CHH
            pltpu.sync_copy(srcs_hbm.at[pl.ds(base, CHH)], src_idx)
            pltpu.sync_copy(dsts_hbm.at[pl.ds(base, CHH)], dst_idx)
            for b in range(2):
                start_gather(b, b)
            if half == 0:
                pltpu.sync_copy(zeros2d_hbm, acc.at[pl.ds(sid * RPT, RPT)])
                pltpu.sync_copy(zeros1d_hbm, dacc.at[pl.ds(sid * RPT, RPT)])
                pltpu.sync_copy(ones1d_hbm, ones_v)
                plsc.subcore_barrier()

            def body(g, carry):
                for b in range(2):
                    c = 2 * g + b
                    wait_gather(c, b)
                    pltpu.sync_copy(rows[b], acc.at[dst_idx.at[c]], add=True)
                    pltpu.sync_copy(ones_v, dacc.at[dst_idx.at[c]], add=True)
                    start_gather(c + 2, b)
                return carry

            lax.fori_loop(0, CHH // 2 - 1, body, 0)
            for b in range(2):
                c = CHH - 2 + b
                wait_gather(c, b)
                pltpu.sync_copy(rows[b], acc.at[dst_idx.at[c]], add=True)
                pltpu.sync_copy(ones_v, dacc.at[dst_idx.at[c]], add=True)
        plsc.subcore_barrier()
        pltpu.sync_copy(acc.at[pl.ds(sid * RPT, RPT)],
                        out_hbm.at[cid, pl.ds(sid * RPT, RPT)])
        pltpu.sync_copy(dacc.at[pl.ds(sid * RPT, RPT)],
                        deg_hbm.at[pl.ds(cid * NPAD + sid * RPT, RPT)])

    return agg_kernel(table, srcs, dsts, zeros2d, zeros1d, ones1d)


def _tc_stage(feats, p0, p1, d0, d1, w_self, w_neigh, b_sage,
              w1, b1, g1, be1, w2, b2, g2, be2, w3, b3):
    blk = 800
    nblk = -(-N // blk)
    npg = N // G
    gpb = blk // npg

    def body(f_ref, p0_ref, p1_ref, d0_ref, d1_ref, ws_ref, wn_ref, b_ref,
             w1_ref, b1_ref, g1_ref, be1_ref, w2_ref, b2_ref, g2_ref,
             be2_ref, w3_ref, b3_ref, o_ref, hg_s):
        i = pl.program_id(0)

        @pl.when(i < nblk)
        def _node():
            agg = p0_ref[...] + p1_ref[...]
            deg = jnp.maximum(d0_ref[...] + d1_ref[...], 1.0)
            h_neigh = agg / deg
            t = jnp.dot(h_neigh, wn_ref[...],
                        preferred_element_type=jnp.float32)
            h = (jnp.dot(f_ref[...], ws_ref[...],
                         preferred_element_type=jnp.float32)
                 + t + b_ref[...])
            h = jnp.maximum(h, 0.0)
            hg_s[pl.ds(i * gpb, gpb), :] = jnp.max(
                h.reshape(gpb, npg, D), axis=1)

        @pl.when(i == nblk)
        def _head():
            hg = hg_s[...][:G, :]
            z = jnp.maximum(
                jnp.dot(hg, w1_ref[...], preferred_element_type=jnp.float32)
                + b1_ref[...], 0.0)
            m = jnp.mean(z, axis=0, keepdims=True)
            v = jnp.mean((z - m) ** 2, axis=0, keepdims=True)
            z = g1_ref[...] * (z - m) * lax.rsqrt(v + 1e-5) + be1_ref[...]
            z = jnp.maximum(
                jnp.dot(z, w2_ref[...], preferred_element_type=jnp.float32)
                + b2_ref[...], 0.0)
            m = jnp.mean(z, axis=0, keepdims=True)
            v = jnp.mean((z - m) ** 2, axis=0, keepdims=True)
            z = g2_ref[...] * (z - m) * lax.rsqrt(v + 1e-5) + be2_ref[...]
            o_ref[...] = (jnp.dot(z, w3_ref[...],
                                  preferred_element_type=jnp.float32)
                          + b3_ref[...])

    clamp = lambda i: (jnp.minimum(i, nblk - 1), 0)
    const = lambda i: (0, 0)
    return pl.pallas_call(
        body,
        grid=(nblk + 1,),
        in_specs=[
            pl.BlockSpec((blk, D), clamp),
            pl.BlockSpec((blk, D), clamp),
            pl.BlockSpec((blk, D), clamp),
            pl.BlockSpec((blk, 1), clamp),
            pl.BlockSpec((blk, 1), clamp),
            pl.BlockSpec((D, D), const),
            pl.BlockSpec((D, D), const),
            pl.BlockSpec((1, D), const),
            pl.BlockSpec((D, D), const),
            pl.BlockSpec((1, D), const),
            pl.BlockSpec((1, D), const),
            pl.BlockSpec((1, D), const),
            pl.BlockSpec((D, 64), const),
            pl.BlockSpec((1, 64), const),
            pl.BlockSpec((1, 64), const),
            pl.BlockSpec((1, 64), const),
            pl.BlockSpec((64, 1), const),
            pl.BlockSpec((1, 1), const),
        ],
        out_specs=pl.BlockSpec((G, 1), const),
        out_shape=jax.ShapeDtypeStruct((G, 1), jnp.float32),
        scratch_shapes=[pltpu.VMEM((nblk * gpb, D), jnp.float32)],
    )(feats, p0, p1, d0, d1, w_self, w_neigh, b_sage,
      w1, b1, g1, be1, w2, b2, g2, be2, w3, b3)


def kernel(feats, edge_index, node_graph_ids, W_self, W_neigh, b_sage,
           W1, b1, g1, be1, W2, b2, g2, be2, W3, b3):
    src = edge_index[0]
    dst = edge_index[1]
    pad_i = jnp.arange(EPAD - E, dtype=jnp.int32)
    srcs = jnp.concatenate(
        [src, pad_i % N]).reshape(NW * CH, CHUNK)
    dsts = jnp.concatenate(
        [dst, N + pad_i % (NPAD - N)]).reshape(NW * CH, CHUNK)
    zeros2d = jnp.zeros((RPT, D), jnp.float32)
    zeros1d = jnp.zeros((RPT,), jnp.float32)
    ones1d = jnp.ones((CHUNK,), jnp.float32)

    parts, degs = _sc_aggregate(feats, srcs, dsts, zeros2d, zeros1d, ones1d)

    d2 = degs.reshape(2, NPAD, 1)
    return _tc_stage(feats, parts[0], parts[1], d2[0], d2[1], W_self, W_neigh,
                     b_sage.reshape(1, D), W1, b1.reshape(1, -1),
                     g1.reshape(1, -1), be1.reshape(1, -1), W2,
                     b2.reshape(1, -1), g2.reshape(1, -1), be2.reshape(1, -1),
                     W3, b3.reshape(1, -1))
